# Initial kernel scaffold; baseline (speedup 1.0000x reference)
import jax
import jax.numpy as jnp
from jax.experimental import pallas as pl


def kernel(x, position_ids, inv_freq):
    raise NotImplementedError("write your pallas kernel here")


# speedup vs baseline: 2.1693x; 2.1693x over previous
import functools

import jax
import jax.numpy as jnp
from jax import lax
from jax.experimental import pallas as pl
from jax.experimental.pallas import tpu as pltpu
from jax.experimental.pallas import tpu_sc as plsc

V = 1024
D = 64
B = 16 * 4096
NC, NS = 2, 16
NW = NC * NS
IDS_PER_ROW = 128
ROWS_TOTAL = B // IDS_PER_ROW
ROWS_PER_W = ROWS_TOTAL // NW
ROWS_PER_CHUNK = 4
CHUNKS = ROWS_PER_W // ROWS_PER_CHUNK
IDS_PER_CHUNK = ROWS_PER_CHUNK * IDS_PER_ROW


def _tables_body(inv_ref, cos_ref, sin_ref):
    f = inv_ref[...]
    cos_ref[...] = jnp.cos(f)
    sin_ref[...] = jnp.sin(f)


def _make_tables(inv_freq):
    return pl.pallas_call(
        _tables_body,
        out_shape=(
            jax.ShapeDtypeStruct((V, D), jnp.float32),
            jax.ShapeDtypeStruct((V, D), jnp.float32),
        ),
    )(inv_freq)


def _gather_body(cos_tab, sin_tab, idx_hbm, cos_out, sin_out,
                 idx_v, cos_buf, sin_buf, sem):
    wid = lax.axis_index("s") * NC + lax.axis_index("c")

    def chunk_body(c, carry):
        row0 = wid * ROWS_PER_W + c * ROWS_PER_CHUNK
        pltpu.sync_copy(idx_hbm.at[pl.ds(row0, ROWS_PER_CHUNK)], idx_v)
        cps = []
        for j in range(ROWS_PER_CHUNK):
            dst = pl.ds(j * IDS_PER_ROW, IDS_PER_ROW)
            cps.append(pltpu.async_copy(
                cos_tab.at[idx_v.at[j]], cos_buf.at[dst], sem))
            cps.append(pltpu.async_copy(
                sin_tab.at[idx_v.at[j]], sin_buf.at[dst], sem))
        for cp in cps:
            cp.wait()
        base = row0 * IDS_PER_ROW
        pltpu.sync_copy(cos_buf, cos_out.at[pl.ds(base, IDS_PER_CHUNK)])
        pltpu.sync_copy(sin_buf, sin_out.at[pl.ds(base, IDS_PER_CHUNK)])
        return carry

    lax.fori_loop(0, CHUNKS, chunk_body, 0)


@functools.cache
def _make_gather():
    return pl.kernel(
        _gather_body,
        out_type=(
            jax.ShapeDtypeStruct((B, D), jnp.float32),
            jax.ShapeDtypeStruct((B, D), jnp.float32),
        ),
        mesh=plsc.VectorSubcoreMesh(core_axis_name="c", subcore_axis_name="s"),
        compiler_params=pltpu.CompilerParams(use_tc_tiling_on_sc=False),
        scratch_types=[
            pltpu.VMEM((ROWS_PER_CHUNK, IDS_PER_ROW), jnp.int32),
            pltpu.VMEM((IDS_PER_CHUNK, D), jnp.float32),
            pltpu.VMEM((IDS_PER_CHUNK, D), jnp.float32),
            pltpu.SemaphoreType.DMA,
        ],
    )


def kernel(x, position_ids, inv_freq):
    cos_tab, sin_tab = _make_tables(inv_freq.astype(jnp.float32))
    idx = position_ids.reshape(ROWS_TOTAL, IDS_PER_ROW).astype(jnp.int32)
    cos_f, sin_f = _make_gather()(cos_tab, sin_tab, idx)
    shape = position_ids.shape + (D,)
    return (cos_f.reshape(shape).astype(x.dtype),
            sin_f.reshape(shape).astype(x.dtype))
